# baseline (speedup 1.0000x reference)
import functools
import math

import jax
import jax.numpy as jnp
from jax import lax
from jax.experimental import pallas as pl
from jax.experimental.pallas import tpu as pltpu
from jax.experimental.pallas import tpu_sc as plsc


def _make_sc_kernel(N, D, scale):
    info = plsc.get_sparse_core_info()
    NC, NS, L = info.num_cores, info.num_subcores, info.num_lanes
    NW = NC * NS
    per_w = N // NW
    C = 16
    n_chunks = per_w // C
    NB = 6
    LA = 3
    mesh = plsc.VectorSubcoreMesh(core_axis_name="c", subcore_axis_name="s")

    @functools.partial(
        pl.kernel,
        mesh=mesh,
        out_type=jax.ShapeDtypeStruct((N, D), jnp.float32),
        scratch_types=(
            [pltpu.VMEM((n_chunks, C), jnp.int32)]
            + [pltpu.VMEM((C, D), jnp.float32) for _ in range(NB)]
            + [pltpu.SemaphoreType.DMA((NB,)),
               pltpu.SemaphoreType.DMA((NB,))]
        ),
    )
    def k(x_hbm, lut_hbm, out_hbm, idx_all, *rest):
        rows = rest[:NB]
        gsem, ssem = rest[NB], rest[NB + 1]
        wid = lax.axis_index("s") * NC + lax.axis_index("c")
        base = wid * per_w

        pltpu.sync_copy(x_hbm.at[wid], idx_all)

        def gather(c):
            b = c % NB
            return pltpu.async_copy(
                lut_hbm.at[idx_all.at[c]], rows[b], gsem.at[b])

        def scatter(c):
            b = c % NB
            return pltpu.async_copy(
                rows[b], out_hbm.at[pl.ds(base + c * C, C), :], ssem.at[b])

        def scale_buf(rv):
            def row_body(r, carry):
                for j in range(D // L):
                    sl = pl.ds(j * L, L)
                    rv[r, sl] = rv[r, sl] * scale
                return carry
            lax.fori_loop(0, C, row_body, 0)

        h_g = [None] * n_chunks
        h_s = [None] * n_chunks
        for c in range(LA):
            h_g[c] = gather(c)
        for c in range(n_chunks):
            if c + LA < n_chunks:
                if c + LA - NB >= 0:
                    h_s[c + LA - NB].wait()
                h_g[c + LA] = gather(c + LA)
            h_g[c].wait()
            scale_buf(rows[c % NB])
            h_s[c] = scatter(c)
        for c in range(n_chunks - NB, n_chunks):
            h_s[c].wait()

    return k


def kernel(x, lut):
    B, S = x.shape
    _, D = lut.shape
    N = B * S
    info = plsc.get_sparse_core_info()
    NW = info.num_cores * info.num_subcores
    per_w = N // NW
    C = 16
    scale = float(math.sqrt(D))
    xf = x.reshape(NW, per_w // C, C).astype(jnp.int32)
    out = _make_sc_kernel(N, D, scale)(xf, lut)
    return out.reshape(B, S, D)
